# baseline (speedup 1.0000x reference)
import functools

import jax
import jax.numpy as jnp
from jax import lax
from jax.experimental import pallas as pl
from jax.experimental.pallas import tpu as pltpu
from jax.experimental.pallas import tpu_sc as plsc

SEQ = 10
LOC = 20
ED = 128
HALF = 64
BATCH = 16384

ROWS = BATCH * SEQ
NC, NS = 2, 16
NW = NC * NS
RPW = ROWS // NW
CHUNK = 128
NCH = RPW // CHUNK
NPAIR = 2 * LOC * LOC // 2


def _table_body(tok_ref, wt_ref, b_ref, pos_ref, o_ref):
    t = lax.dot_general(tok_ref[:], wt_ref[:], (((1,), (1,)), ((), ())),
                        preferred_element_type=jnp.float32)
    t = t + b_ref[:]
    zeros = jnp.zeros((LOC, HALF), jnp.float32)
    tl = jnp.concatenate([t, zeros], axis=1)
    tr = jnp.concatenate([zeros, t], axis=1)
    poshdup = jnp.concatenate([pos_ref[:], pos_ref[:]], axis=1)
    q_iota = lax.broadcasted_iota(jnp.int32, (NPAIR, LOC), 0)
    k_iota = lax.broadcasted_iota(jnp.int32, (NPAIR, LOC), 1)
    oh_l = ((2 * q_iota) % LOC == k_iota).astype(jnp.float32)
    oh_r = ((2 * q_iota + 1) % LOC == k_iota).astype(jnp.float32)
    oh_m = (q_iota // SEQ == k_iota).astype(jnp.float32)
    dims = (((1,), (0,)), ((), ()))
    o_ref[:] = (
        lax.dot_general(oh_l, tl, dims, preferred_element_type=jnp.float32)
        + lax.dot_general(oh_r, tr, dims, preferred_element_type=jnp.float32)
        + lax.dot_general(oh_m, poshdup, dims,
                          preferred_element_type=jnp.float32))


def _build_table(tok_table, W, b, pos_table):
    return pl.pallas_call(
        _table_body,
        out_shape=jax.ShapeDtypeStruct((NPAIR, ED), jnp.float32),
    )(tok_table, jnp.transpose(W), b.reshape(1, HALF),
      pos_table.reshape(2 * SEQ, HALF))


def _gather_body(h_hbm, q_hbm, out_hbm, hbuf, qbuf, p0buf, p1buf, buf0, buf1,
                 ssem0, ssem1, hsem):
    wid = lax.axis_index("s") * NC + lax.axis_index("c")
    rowbase = wid * RPW

    pltpu.make_async_copy(h_hbm, hbuf, hsem).start()
    pltpu.sync_copy(q_hbm.at[pl.ds(rowbase * 2, RPW * 2)], qbuf)

    def idx_body(g, carry):
        base = (g // 8) * 256 + (g % 8) * 16
        i0 = qbuf[pl.ds(base, 16)]
        i1 = qbuf[pl.ds(base + 128, 16)]
        s40 = ((rowbase + g * 16) // BATCH) * (2 * LOC)
        p0buf[pl.ds(g * 16, 16)] = s40 + i0
        p1buf[pl.ds(g * 16, 16)] = (s40 + LOC) + i1
        return carry

    lax.fori_loop(0, RPW // 16, idx_body, 0)
    pltpu.make_async_copy(h_hbm, hbuf, hsem).wait()

    lanes = lax.iota(jnp.int32, 16)

    def assemble_chunk(t, buf):
        def group_body(g, carry):
            p0 = p0buf[pl.ds(t * CHUNK + g * 16, 16)]
            p1 = p1buf[pl.ds(t * CHUNK + g * 16, 16)]
            r0 = p0 >> 1
            c0 = (p0 & 1) << 6
            r1 = p1 >> 1
            c1 = (p1 & 1) << 6
            dst = g * 16 + lanes

            def k_body(k8, carry):
                for kk in range(8):
                    k = k8 * 8 + kk
                    vl = plsc.load_gather(hbuf, [r0, c0 + k])
                    plsc.store_scatter(buf, [dst, jnp.full((16,), k,
                                                           jnp.int32)], vl)
                    vr = plsc.load_gather(hbuf, [r1, c1 + k])
                    plsc.store_scatter(buf, [dst, jnp.full((16,), HALF + k,
                                                           jnp.int32)], vr)
                return carry

            lax.fori_loop(0, HALF // 8, k_body, 0)
            return carry

        lax.fori_loop(0, CHUNK // 16, group_body, 0)

    def start_scatter(t, buf, sem):
        pltpu.make_async_copy(
            buf, out_hbm.at[pl.ds(rowbase + t * CHUNK, CHUNK)], sem).start()

    def wait_scatter(buf, sem):
        pltpu.make_async_copy(
            buf, out_hbm.at[pl.ds(rowbase, CHUNK)], sem).wait()

    def pipe_body(t2, carry):
        t = t2 * 2
        for j, (buf, sem) in enumerate(((buf0, ssem0), (buf1, ssem1))):
            tt = t + j

            @pl.when(tt >= 2)
            def _():
                wait_scatter(buf, sem)

            assemble_chunk(tt, buf)
            start_scatter(tt, buf, sem)
        return carry

    lax.fori_loop(0, NCH // 2, pipe_body, 0)
    wait_scatter(buf0, ssem0)
    wait_scatter(buf1, ssem1)


def _gather_rows(h_packed, q_flat):
    mesh = plsc.VectorSubcoreMesh(core_axis_name="c", subcore_axis_name="s")
    f = functools.partial(
        pl.kernel,
        mesh=mesh,
        compiler_params=pltpu.CompilerParams(needs_layout_passes=False),
        out_type=jax.ShapeDtypeStruct((ROWS, ED), jnp.float32),
        scratch_types=[
            pltpu.VMEM((NPAIR, ED), jnp.float32),
            pltpu.VMEM((2 * RPW,), jnp.int32),
            pltpu.VMEM((RPW,), jnp.int32),
            pltpu.VMEM((RPW,), jnp.int32),
            pltpu.VMEM((CHUNK, ED), jnp.float32),
            pltpu.VMEM((CHUNK, ED), jnp.float32),
            pltpu.SemaphoreType.DMA,
            pltpu.SemaphoreType.DMA,
            pltpu.SemaphoreType.DMA,
        ],
    )(_gather_body)
    return f(h_packed, q_flat)


def kernel(inputs, tok_table, W, b, pos_table):
    h = _build_table(tok_table, W, b, pos_table)
    q = jnp.transpose(
        inputs.astype(jnp.int32).reshape(BATCH // 128, 128, SEQ, 2),
        (2, 0, 3, 1),
    ).reshape(-1)
    out = _gather_rows(h, q)
    return jnp.transpose(out.reshape(SEQ, BATCH, ED), (1, 0, 2))


# speedup vs baseline: 13.1427x; 13.1427x over previous
import functools

import jax
import jax.numpy as jnp
from jax import lax
from jax.experimental import pallas as pl
from jax.experimental.pallas import tpu as pltpu
from jax.experimental.pallas import tpu_sc as plsc

SEQ = 10
LOC = 20
ED = 128
HALF = 64
BATCH = 16384

ROWS = BATCH * SEQ
NC, NS = 2, 16
NW = NC * NS
RPW = ROWS // NW
CHUNK = 128
NCH = RPW // CHUNK


def _table_body(tok_ref, wt_ref, b_ref, pos_ref, o_ref):
    t = lax.dot_general(tok_ref[:], wt_ref[:], (((1,), (1,)), ((), ())),
                        preferred_element_type=jnp.float32)
    zeros = jnp.zeros((LOC, HALF), jnp.float32)
    tl = jnp.concatenate([t, zeros], axis=1)
    tr = jnp.concatenate([zeros, t], axis=1)
    p_iota = lax.broadcasted_iota(jnp.int32, (LOC * LOC, LOC), 0)
    k_iota = lax.broadcasted_iota(jnp.int32, (LOC * LOC, LOC), 1)
    onehot = (p_iota // LOC == k_iota).astype(jnp.float32)
    s400 = lax.dot_general(onehot, tl, (((1,), (0,)), ((), ())),
                           preferred_element_type=jnp.float32)
    s400 = s400 + jnp.concatenate([tr] * LOC, axis=0)
    posb = pos_ref[:] + jnp.concatenate([b_ref[:], b_ref[:]], axis=1)
    o_ref[:] = s400[None, :, :] + posb[:, None, :]


def _build_table(tok_table, W, b, pos_table):
    return pl.pallas_call(
        _table_body,
        out_shape=jax.ShapeDtypeStruct((SEQ, LOC * LOC, ED), jnp.float32),
    )(tok_table, jnp.transpose(W), b.reshape(1, HALF), pos_table)


def _gather_body(g_hbm, q_hbm, out_hbm, gshared, qbuf, cidx, buf0, buf1,
                 buf2, buf3,
                 gsem0, gsem1, gsem2, gsem3, ssem0, ssem1, ssem2, ssem3):
    sid = lax.axis_index("s")
    wid = sid * NC + lax.axis_index("c")
    rowbase = wid * RPW

    @pl.when(sid == 0)
    def _():
        pltpu.sync_copy(g_hbm, gshared)

    pltpu.sync_copy(q_hbm.at[pl.ds(rowbase * 2, RPW * 2)], qbuf)

    def idx_body(g, carry):
        base = (g // 8) * 256 + (g % 8) * 16
        i0 = qbuf[pl.ds(base, 16)]
        i1 = qbuf[pl.ds(base + 128, 16)]
        s = (rowbase + g * 16) // BATCH
        c = s * (LOC * LOC) + i0 * LOC + i1
        cidx[g // 8, pl.ds((g % 8) * 16, 16)] = c
        return carry

    lax.fori_loop(0, RPW // 16, idx_body, 0)
    plsc.subcore_barrier()

    def start_gather(t, buf, sem):
        pltpu.make_async_copy(gshared.at[cidx.at[t]], buf, sem).start()

    def wait_gather(buf, sem):
        pltpu.make_async_copy(gshared.at[cidx.at[0]], buf, sem).wait()

    def start_scatter(t, buf, sem):
        pltpu.make_async_copy(
            buf, out_hbm.at[pl.ds(rowbase + t * CHUNK, CHUNK)], sem).start()

    def wait_scatter(buf, sem):
        pltpu.make_async_copy(
            buf, out_hbm.at[pl.ds(rowbase, CHUNK)], sem).wait()

    bufs = [(buf0, gsem0, ssem0), (buf1, gsem1, ssem1),
            (buf2, gsem2, ssem2), (buf3, gsem3, ssem3)]

    start_gather(0, buf0, gsem0)
    start_gather(1, buf1, gsem1)

    def pipe_body(t4, carry):
        for j in range(4):
            t = t4 * 4 + j
            buf, gsem, ssem = bufs[j]
            nbuf, ngsem, nssem = bufs[(j + 2) % 4]
            wait_gather(buf, gsem)
            start_scatter(t, buf, ssem)

            @pl.when(t < NCH - 2)
            def _():
                @pl.when(t >= 2)
                def _():
                    wait_scatter(nbuf, nssem)

                start_gather(t + 2, nbuf, ngsem)

        return carry

    lax.fori_loop(0, NCH // 4, pipe_body, 0)
    for j in range(4):
        buf, _, ssem = bufs[j]
        wait_scatter(buf, ssem)


def _gather_rows(g_flat, q_flat):
    mesh = plsc.VectorSubcoreMesh(core_axis_name="c", subcore_axis_name="s")
    f = functools.partial(
        pl.kernel,
        mesh=mesh,
        out_type=jax.ShapeDtypeStruct((ROWS, ED), jnp.float32),
        scratch_types=[
            pltpu.VMEM_SHARED((SEQ * LOC * LOC, ED), jnp.float32),
            pltpu.VMEM((2 * RPW,), jnp.int32),
            pltpu.VMEM((NCH, CHUNK), jnp.int32),
            pltpu.VMEM((CHUNK, ED), jnp.float32),
            pltpu.VMEM((CHUNK, ED), jnp.float32),
            pltpu.VMEM((CHUNK, ED), jnp.float32),
            pltpu.VMEM((CHUNK, ED), jnp.float32),
        ] + [pltpu.SemaphoreType.DMA] * 8,
    )(_gather_body)
    return f(g_flat, q_flat)


def kernel(inputs, tok_table, W, b, pos_table):
    g = _build_table(tok_table, W, b, pos_table).reshape(SEQ * LOC * LOC, ED)
    q = jnp.transpose(
        inputs.astype(jnp.int32).reshape(BATCH // 128, 128, SEQ, 2),
        (2, 0, 3, 1),
    ).reshape(-1)
    out = _gather_rows(g, q)
    return jnp.transpose(out.reshape(SEQ, BATCH, ED), (1, 0, 2))


# speedup vs baseline: 13.8431x; 1.0533x over previous
import functools

import jax
import jax.numpy as jnp
from jax import lax
from jax.experimental import pallas as pl
from jax.experimental.pallas import tpu as pltpu
from jax.experimental.pallas import tpu_sc as plsc

SEQ = 10
LOC = 20
ED = 128
HALF = 64
BATCH = 16384

ROWS = BATCH * SEQ
NC, NS = 2, 16
NW = NC * NS
RPW = ROWS // NW
CHUNK = 128
NCH = RPW // CHUNK


def _table_body(tok_ref, wt_ref, b_ref, pos_ref, o_ref):
    t = lax.dot_general(tok_ref[:], wt_ref[:], (((1,), (1,)), ((), ())),
                        preferred_element_type=jnp.float32)
    zeros = jnp.zeros((LOC, HALF), jnp.float32)
    tl = jnp.concatenate([t, zeros], axis=1)
    tr = jnp.concatenate([zeros, t], axis=1)
    p_iota = lax.broadcasted_iota(jnp.int32, (LOC * LOC, LOC), 0)
    k_iota = lax.broadcasted_iota(jnp.int32, (LOC * LOC, LOC), 1)
    onehot = (p_iota // LOC == k_iota).astype(jnp.float32)
    s400 = lax.dot_general(onehot, tl, (((1,), (0,)), ((), ())),
                           preferred_element_type=jnp.float32)
    s400 = s400 + jnp.concatenate([tr] * LOC, axis=0)
    posb = pos_ref[:] + jnp.concatenate([b_ref[:], b_ref[:]], axis=1)
    o_ref[:] = s400[None, :, :] + posb[:, None, :]


def _build_table(tok_table, W, b, pos_table):
    return pl.pallas_call(
        _table_body,
        out_shape=jax.ShapeDtypeStruct((SEQ, LOC * LOC, ED), jnp.float32),
    )(tok_table, jnp.transpose(W), b.reshape(1, HALF), pos_table)


def _gather_body(g_hbm, q_hbm, out_hbm, gshared, qbuf, cidx, buf0, buf1,
                 buf2, buf3,
                 gsem0, gsem1, gsem2, gsem3, ssem0, ssem1, ssem2, ssem3,
                 stagesem):
    sid = lax.axis_index("s")
    wid = sid * NC + lax.axis_index("c")
    rowbase = wid * RPW

    @pl.when(sid == 0)
    def _():
        pltpu.make_async_copy(g_hbm, gshared, stagesem).start()

    pltpu.sync_copy(q_hbm.at[pl.ds(rowbase * 2, RPW * 2)], qbuf)

    def idx_body(g, carry):
        base = (g // 8) * 256 + (g % 8) * 16
        i0 = qbuf[pl.ds(base, 16)]
        i1 = qbuf[pl.ds(base + 128, 16)]
        s = (rowbase + g * 16) // BATCH
        c = s * (LOC * LOC) + i0 * LOC + i1
        cidx[g // 8, pl.ds((g % 8) * 16, 16)] = c
        return carry

    def compute_cidx(t):
        lax.fori_loop(t * 8, t * 8 + 8, idx_body, 0)

    compute_cidx(0)
    compute_cidx(1)

    @pl.when(sid == 0)
    def _():
        pltpu.make_async_copy(g_hbm, gshared, stagesem).wait()

    plsc.subcore_barrier()

    def start_gather(t, buf, sem):
        pltpu.make_async_copy(gshared.at[cidx.at[t]], buf, sem).start()

    def wait_gather(buf, sem):
        pltpu.make_async_copy(gshared.at[cidx.at[0]], buf, sem).wait()

    def start_scatter(t, buf, sem):
        pltpu.make_async_copy(
            buf, out_hbm.at[pl.ds(rowbase + t * CHUNK, CHUNK)], sem).start()

    def wait_scatter(buf, sem):
        pltpu.make_async_copy(
            buf, out_hbm.at[pl.ds(rowbase, CHUNK)], sem).wait()

    bufs = [(buf0, gsem0, ssem0), (buf1, gsem1, ssem1),
            (buf2, gsem2, ssem2), (buf3, gsem3, ssem3)]

    start_gather(0, buf0, gsem0)
    start_gather(1, buf1, gsem1)

    def pipe_body(t4, carry):
        for j in range(4):
            t = t4 * 4 + j
            buf, gsem, ssem = bufs[j]
            nbuf, ngsem, nssem = bufs[(j + 2) % 4]
            wait_gather(buf, gsem)
            start_scatter(t, buf, ssem)

            @pl.when(t < NCH - 2)
            def _():
                compute_cidx(t + 2)

                @pl.when(t >= 2)
                def _():
                    wait_scatter(nbuf, nssem)

                start_gather(t + 2, nbuf, ngsem)

        return carry

    lax.fori_loop(0, NCH // 4, pipe_body, 0)
    for j in range(4):
        buf, _, ssem = bufs[j]
        wait_scatter(buf, ssem)


def _gather_rows(g_flat, q_flat):
    mesh = plsc.VectorSubcoreMesh(core_axis_name="c", subcore_axis_name="s")
    f = functools.partial(
        pl.kernel,
        mesh=mesh,
        out_type=jax.ShapeDtypeStruct((ROWS, ED), jnp.float32),
        scratch_types=[
            pltpu.VMEM_SHARED((SEQ * LOC * LOC, ED), jnp.float32),
            pltpu.VMEM((2 * RPW,), jnp.int32),
            pltpu.VMEM((NCH, CHUNK), jnp.int32),
            pltpu.VMEM((CHUNK, ED), jnp.float32),
            pltpu.VMEM((CHUNK, ED), jnp.float32),
            pltpu.VMEM((CHUNK, ED), jnp.float32),
            pltpu.VMEM((CHUNK, ED), jnp.float32),
        ] + [pltpu.SemaphoreType.DMA] * 9,
    )(_gather_body)
    return f(g_flat, q_flat)


def kernel(inputs, tok_table, W, b, pos_table):
    g = _build_table(tok_table, W, b, pos_table).reshape(SEQ * LOC * LOC, ED)
    q = jnp.transpose(
        inputs.astype(jnp.int32).reshape(BATCH // 128, 128, SEQ, 2),
        (2, 0, 3, 1),
    ).reshape(-1)
    out = _gather_rows(g, q)
    return jnp.transpose(out.reshape(SEQ, BATCH, ED), (1, 0, 2))


# speedup vs baseline: 14.0171x; 1.0126x over previous
import functools

import jax
import jax.numpy as jnp
from jax import lax
from jax.experimental import pallas as pl
from jax.experimental.pallas import tpu as pltpu
from jax.experimental.pallas import tpu_sc as plsc

SEQ = 10
LOC = 20
ED = 128
HALF = 64
BATCH = 16384

ROWS = BATCH * SEQ
NC, NS = 2, 16
NW = NC * NS
RPW = ROWS // NW
CHUNK = 128
NCH = RPW // CHUNK


def _table_body(tok_ref, wt_ref, b_ref, pos_ref, o_ref):
    t = lax.dot_general(tok_ref[:], wt_ref[:], (((1,), (1,)), ((), ())),
                        preferred_element_type=jnp.float32)
    zeros = jnp.zeros((LOC, HALF), jnp.float32)
    tl = jnp.concatenate([t, zeros], axis=1)
    tr = jnp.concatenate([zeros, t], axis=1)
    p_iota = lax.broadcasted_iota(jnp.int32, (LOC * LOC, LOC), 0)
    k_iota = lax.broadcasted_iota(jnp.int32, (LOC * LOC, LOC), 1)
    onehot = (p_iota // LOC == k_iota).astype(jnp.float32)
    s400 = lax.dot_general(onehot, tl, (((1,), (0,)), ((), ())),
                           preferred_element_type=jnp.float32)
    s400 = s400 + jnp.concatenate([tr] * LOC, axis=0)
    posb = pos_ref[:] + jnp.concatenate([b_ref[:], b_ref[:]], axis=1)
    o_ref[:] = s400[None, :, :] + posb[:, None, :]


def _build_table(tok_table, W, b, pos_table):
    return pl.pallas_call(
        _table_body,
        out_shape=jax.ShapeDtypeStruct((SEQ, LOC * LOC, ED), jnp.float32),
    )(tok_table, jnp.transpose(W), b.reshape(1, HALF), pos_table)


def _gather_body(g_hbm, q_hbm, out_hbm, gshared, qbuf, cidx, buf0, buf1,
                 buf2, buf3, buf4,
                 gsem0, gsem1, gsem2, gsem3, gsem4,
                 ssem0, ssem1, ssem2, ssem3, ssem4,
                 stagesem):
    sid = lax.axis_index("s")
    wid = sid * NC + lax.axis_index("c")
    rowbase = wid * RPW

    @pl.when(sid == 0)
    def _():
        pltpu.make_async_copy(g_hbm, gshared, stagesem).start()

    pltpu.sync_copy(q_hbm.at[pl.ds(rowbase * 2, RPW * 2)], qbuf)

    def idx_body(g, carry):
        base = (g // 8) * 256 + (g % 8) * 16
        i0 = qbuf[pl.ds(base, 16)]
        i1 = qbuf[pl.ds(base + 128, 16)]
        s = (rowbase + g * 16) // BATCH
        c = s * (LOC * LOC) + i0 * LOC + i1
        cidx[g // 8, pl.ds((g % 8) * 16, 16)] = c
        return carry

    def compute_cidx(t):
        lax.fori_loop(t * 8, t * 8 + 8, idx_body, 0)

    compute_cidx(0)
    compute_cidx(1)

    @pl.when(sid == 0)
    def _():
        pltpu.make_async_copy(g_hbm, gshared, stagesem).wait()

    plsc.subcore_barrier()

    def start_gather(t, buf, sem):
        pltpu.make_async_copy(gshared.at[cidx.at[t]], buf, sem).start()

    def wait_gather(buf, sem):
        pltpu.make_async_copy(gshared.at[cidx.at[0]], buf, sem).wait()

    def start_scatter(t, buf, sem):
        pltpu.make_async_copy(
            buf, out_hbm.at[pl.ds(rowbase + t * CHUNK, CHUNK)], sem).start()

    def wait_scatter(buf, sem):
        pltpu.make_async_copy(
            buf, out_hbm.at[pl.ds(rowbase, CHUNK)], sem).wait()

    bufs = [(buf0, gsem0, ssem0), (buf1, gsem1, ssem1),
            (buf2, gsem2, ssem2), (buf3, gsem3, ssem3),
            (buf4, gsem4, ssem4)]

    compute_cidx(2)
    start_gather(0, buf0, gsem0)
    start_gather(1, buf1, gsem1)
    start_gather(2, buf2, gsem2)

    def pipe_body(t5, carry):
        for j in range(5):
            t = t5 * 5 + j
            buf, gsem, ssem = bufs[j]
            nbuf, ngsem, nssem = bufs[(j + 3) % 5]

            @pl.when(t < NCH - 3)
            def _():
                compute_cidx(t + 3)

            wait_gather(buf, gsem)
            start_scatter(t, buf, ssem)

            @pl.when(t < NCH - 3)
            def _():
                @pl.when(t >= 2)
                def _():
                    wait_scatter(nbuf, nssem)

                start_gather(t + 3, nbuf, ngsem)

        return carry

    lax.fori_loop(0, NCH // 5, pipe_body, 0)
    for j in range(5):
        buf, _, ssem = bufs[j]
        wait_scatter(buf, ssem)


def _gather_rows(g_flat, q_flat):
    mesh = plsc.VectorSubcoreMesh(core_axis_name="c", subcore_axis_name="s")
    f = functools.partial(
        pl.kernel,
        mesh=mesh,
        out_type=jax.ShapeDtypeStruct((ROWS, ED), jnp.float32),
        scratch_types=[
            pltpu.VMEM_SHARED((SEQ * LOC * LOC, ED), jnp.float32),
            pltpu.VMEM((2 * RPW,), jnp.int32),
            pltpu.VMEM((NCH, CHUNK), jnp.int32),
            pltpu.VMEM((CHUNK, ED), jnp.float32),
            pltpu.VMEM((CHUNK, ED), jnp.float32),
            pltpu.VMEM((CHUNK, ED), jnp.float32),
            pltpu.VMEM((CHUNK, ED), jnp.float32),
            pltpu.VMEM((CHUNK, ED), jnp.float32),
        ] + [pltpu.SemaphoreType.DMA] * 11,
    )(_gather_body)
    return f(g_flat, q_flat)


def kernel(inputs, tok_table, W, b, pos_table):
    g = _build_table(tok_table, W, b, pos_table).reshape(SEQ * LOC * LOC, ED)
    q = jnp.transpose(
        inputs.astype(jnp.int32).reshape(BATCH // 128, 128, SEQ, 2),
        (2, 0, 3, 1),
    ).reshape(-1)
    out = _gather_rows(g, q)
    return jnp.transpose(out.reshape(SEQ, BATCH, ED), (1, 0, 2))
